# Initial kernel scaffold; baseline (speedup 1.0000x reference)
import jax
import jax.numpy as jnp
from jax.experimental import pallas as pl


def kernel(unary_tensor, binary_tensor, emb_W, emb_b, W1, root1, b1, W2, root2, b2):
    raise NotImplementedError("write your pallas kernel here")


# speedup vs baseline: 168.9011x; 168.9011x over previous
import jax
import jax.numpy as jnp
from jax import lax
from jax.experimental import pallas as pl

_T, _B, _N, _FEAT, _R, _EMB = 2, 8, 300, 32, 3, 16
_TB = _T * _B
_NR = _N * _R
_RE = _R * _EMB


def _gnn_kernel(x_ref, adj_ref, embw_ref, embb_ref,
                w1_ref, r1_ref, b1_ref, w2_ref, r2_ref, b2_ref, out_ref):
    af = adj_ref[0].astype(jnp.float32)
    cnt = jnp.sum(af, axis=0, keepdims=True)
    ainv = af * (1.0 / jnp.maximum(cnt, 1.0))

    qi = lax.broadcasted_iota(jnp.int32, (_NR, _RE), 0)
    ci = lax.broadcasted_iota(jnp.int32, (_NR, _RE), 1)
    sel = ((qi % _R) == (ci // _EMB)).astype(jnp.float32)
    fi = lax.broadcasted_iota(jnp.int32, (_RE, _EMB), 0)
    fj = lax.broadcasted_iota(jnp.int32, (_RE, _EMB), 1)
    fold = ((fi % _EMB) == fj).astype(jnp.float32)
    gi = lax.broadcasted_iota(jnp.int32, (_N, _NR), 0)
    gq = lax.broadcasted_iota(jnp.int32, (_N, _NR), 1)
    grp = ((gq // _R) == gi).astype(jnp.float32)

    h = jnp.dot(x_ref[0], embw_ref[...],
                preferred_element_type=jnp.float32) + embb_ref[...]

    def layer(h, wcat, root, bias):
        hw = jnp.dot(h, wcat, preferred_element_type=jnp.float32)
        s2 = lax.dot_general(ainv, hw, (((0,), (0,)), ((), ())),
                             preferred_element_type=jnp.float32)
        y = jnp.dot(s2 * sel, fold, preferred_element_type=jnp.float32)
        z = jnp.dot(grp, y, preferred_element_type=jnp.float32)
        o = jnp.dot(h, root, preferred_element_type=jnp.float32) + bias + z
        return jnp.maximum(o, 0.0)

    h1 = layer(h, w1_ref[...], r1_ref[...], b1_ref[...])
    out_ref[0] = layer(h1, w2_ref[...], r2_ref[...], b2_ref[...])


def kernel(unary_tensor, binary_tensor, emb_W, emb_b, W1, root1, b1, W2, root2, b2):
    x = unary_tensor.astype(jnp.float32).reshape(_TB, _N, _FEAT)
    adj = binary_tensor.reshape(_TB, _N, _NR)
    w1cat = W1.transpose(1, 0, 2).reshape(_EMB, _RE)
    w2cat = W2.transpose(1, 0, 2).reshape(_EMB, _RE)
    b_1 = b1.reshape(1, _EMB)
    b_2 = b2.reshape(1, _EMB)
    ebias = emb_b.reshape(1, _EMB)

    full = lambda *s: pl.BlockSpec(s, lambda b: (0,) * len(s))
    out = pl.pallas_call(
        _gnn_kernel,
        grid=(_TB,),
        in_specs=[
            pl.BlockSpec((1, _N, _FEAT), lambda b: (b, 0, 0)),
            pl.BlockSpec((1, _N, _NR), lambda b: (b, 0, 0)),
            full(_FEAT, _EMB),
            full(1, _EMB),
            full(_EMB, _RE),
            full(_EMB, _EMB),
            full(1, _EMB),
            full(_EMB, _RE),
            full(_EMB, _EMB),
            full(1, _EMB),
        ],
        out_specs=pl.BlockSpec((1, _N, _EMB), lambda b: (b, 0, 0)),
        out_shape=jax.ShapeDtypeStruct((_TB, _N, _EMB), jnp.float32),
    )(x, adj, emb_W, ebias, w1cat, root1, b_1, w2cat, root2, b_2)
    return out.reshape(_TB, _N * _EMB)


# speedup vs baseline: 643.0006x; 3.8070x over previous
import jax
import jax.numpy as jnp
from jax import lax
from jax.experimental import pallas as pl

_T, _B, _N, _FEAT, _R, _EMB = 2, 8, 300, 32, 3, 16


def _dot_t(a, b):
    return lax.dot_general(a, b, (((0,), (0,)), ((), ())),
                           preferred_element_type=jnp.float32)


def _gnn_kernel(xt_ref, adj_ref, embw_ref, embb_ref,
                w1_ref, r1_ref, b1_ref, w2_ref, r2_ref, b2_ref, out_ref):
    for b in range(_B):
        afs = []
        for r in range(_R):
            af = adj_ref[0, :, r, b, :].astype(jnp.float32)
            cnt = jnp.sum(af, axis=0, keepdims=True)
            afs.append(af * (1.0 / jnp.maximum(cnt, 1.0)))

        h = _dot_t(xt_ref[0, b], embw_ref[...]) + embb_ref[...]

        def layer(h, w_ref, root_ref, bias_ref):
            o = jnp.dot(h, root_ref[...],
                        preferred_element_type=jnp.float32) + bias_ref[...]
            for r in range(_R):
                hw = jnp.dot(h, w_ref[r], preferred_element_type=jnp.float32)
                o = o + _dot_t(afs[r], hw)
            return jnp.maximum(o, 0.0)

        h1 = layer(h, w1_ref, r1_ref, b1_ref)
        out_ref[0, b] = layer(h1, w2_ref, r2_ref, b2_ref)


def kernel(unary_tensor, binary_tensor, emb_W, emb_b, W1, root1, b1, W2, root2, b2):
    xt = unary_tensor.astype(jnp.float32).transpose(0, 1, 3, 2)
    adj = binary_tensor.transpose(0, 2, 4, 1, 3)
    full = lambda *s: pl.BlockSpec(s, lambda t: (0,) * len(s))
    out = pl.pallas_call(
        _gnn_kernel,
        grid=(_T,),
        in_specs=[
            pl.BlockSpec((1, _B, _FEAT, _N), lambda t: (t, 0, 0, 0)),
            pl.BlockSpec((1, _N, _R, _B, _N), lambda t: (t, 0, 0, 0, 0)),
            full(_FEAT, _EMB),
            full(1, _EMB),
            full(_R, _EMB, _EMB),
            full(_EMB, _EMB),
            full(1, _EMB),
            full(_R, _EMB, _EMB),
            full(_EMB, _EMB),
            full(1, _EMB),
        ],
        out_specs=pl.BlockSpec((1, _B, _N, _EMB), lambda t: (t, 0, 0, 0)),
        out_shape=jax.ShapeDtypeStruct((_T, _B, _N, _EMB), jnp.float32),
    )(xt, adj, emb_W, emb_b.reshape(1, _EMB), W1, root1, b1.reshape(1, _EMB),
      W2, root2, b2.reshape(1, _EMB))
    return out.reshape(_T * _B, _N * _EMB)


# speedup vs baseline: 800.6572x; 1.2452x over previous
import jax
import jax.numpy as jnp
from jax import lax
from jax.experimental import pallas as pl

_T, _B, _N, _FEAT, _R, _EMB = 2, 8, 300, 32, 3, 16


def _dot_t(a, b):
    return lax.dot_general(a, b, (((0,), (0,)), ((), ())),
                           preferred_element_type=jnp.float32)


def _gnn_kernel(xt_ref, adj_ref, embw_ref, embb_ref,
                w1_ref, r1_ref, b1_ref, w2_ref, r2_ref, b2_ref, out_ref):
    vts = [jnp.swapaxes(adj_ref[0, :, r, :, :], 0, 1).astype(jnp.float32)
           for r in range(_R)]
    ones_col = jnp.ones((_N, 1), jnp.float32)
    for b in range(_B):
        afs = [vts[r][b] for r in range(_R)]
        invs = [1.0 / jnp.maximum(_dot_t(af, ones_col), 1.0) for af in afs]

        h = _dot_t(xt_ref[0, b], embw_ref[...]) + embb_ref[...]

        def layer(h, w_ref, root_ref, bias_ref):
            o = jnp.dot(h, root_ref[...],
                        preferred_element_type=jnp.float32) + bias_ref[...]
            for r in range(_R):
                hw = jnp.dot(h, w_ref[r], preferred_element_type=jnp.float32)
                o = o + _dot_t(afs[r], hw) * invs[r]
            return jnp.maximum(o, 0.0)

        h1 = layer(h, w1_ref, r1_ref, b1_ref)
        out_ref[0, b] = layer(h1, w2_ref, r2_ref, b2_ref)


def kernel(unary_tensor, binary_tensor, emb_W, emb_b, W1, root1, b1, W2, root2, b2):
    xt = unary_tensor.astype(jnp.float32).transpose(0, 1, 3, 2)
    adj = binary_tensor.transpose(0, 2, 4, 1, 3)
    full = lambda *s: pl.BlockSpec(s, lambda t: (0,) * len(s))
    out = pl.pallas_call(
        _gnn_kernel,
        grid=(_T,),
        in_specs=[
            pl.BlockSpec((1, _B, _FEAT, _N), lambda t: (t, 0, 0, 0)),
            pl.BlockSpec((1, _N, _R, _B, _N), lambda t: (t, 0, 0, 0, 0)),
            full(_FEAT, _EMB),
            full(1, _EMB),
            full(_R, _EMB, _EMB),
            full(_EMB, _EMB),
            full(1, _EMB),
            full(_R, _EMB, _EMB),
            full(_EMB, _EMB),
            full(1, _EMB),
        ],
        out_specs=pl.BlockSpec((1, _B, _N, _EMB), lambda t: (t, 0, 0, 0)),
        out_shape=jax.ShapeDtypeStruct((_T, _B, _N, _EMB), jnp.float32),
    )(xt, adj, emb_W, emb_b.reshape(1, _EMB), W1, root1, b1.reshape(1, _EMB),
      W2, root2, b2.reshape(1, _EMB))
    return out.reshape(_T * _B, _N * _EMB)
